# Initial kernel scaffold; baseline (speedup 1.0000x reference)
import jax
import jax.numpy as jnp
from jax.experimental import pallas as pl


def kernel(x, table):
    raise NotImplementedError("write your pallas kernel here")


# speedup vs baseline: 2.0690x; 2.0690x over previous
import functools

import numpy as np
import jax
import jax.numpy as jnp
from jax import lax
from jax.experimental import pallas as pl
from jax.experimental.pallas import tpu as pltpu
from jax.experimental.pallas import tpu_sc as plsc

_VOCAB = 100000
_D = 128
_B = 1024
_S = 200

_NC = 2
_NS = 16
_NW = _NC * _NS
_NROWS = _B * _S
_PER_W = _NROWS // _NW
_CHUNK = 128
_NCHUNK = _PER_W // _CHUNK
_LANES = 16
_GROUPS = _CHUNK // _LANES


def _pos_encoding_2x() -> jax.Array:
    positions = np.arange(_S)
    dimensions = np.arange(_D)
    denominator = np.power(10000.0, 2 * dimensions / _D)
    angles = positions.reshape(-1, 1) / denominator.reshape(1, -1)
    pe = np.zeros(angles.shape)
    pe[:, 0::2] = np.sin(angles[:, 0::2])
    pe[:, 1::2] = np.cos(angles[:, 1::2])
    pe2 = np.concatenate([pe, pe], axis=0)
    return jnp.asarray(pe2, dtype=jnp.float32)


def _emb_body(x_hbm, table_hbm, pe_hbm, out_hbm, idx_v, pe_v, rows_v, gsem):
    wid = lax.axis_index("s") * _NC + lax.axis_index("c")

    pltpu.sync_copy(x_hbm.at[pl.ds(wid * _PER_W, _PER_W)], idx_v)
    pltpu.sync_copy(pe_hbm, pe_v)

    def chunk_body(b, _):
        idx_c = idx_v.at[pl.ds(b * _CHUNK, _CHUNK)]
        pltpu.async_copy(table_hbm.at[idx_c], rows_v, gsem).wait()
        phase = (b * _CHUNK) % _S

        def group_body(g, _):
            rbase = g * _LANES
            for i in range(_LANES):
                r = rbase + i
                rvec = jnp.full((_LANES,), b * _CHUNK + r, dtype=jnp.int32)
                idx_row = plsc.load_gather(idx_v, [rvec])
                keep = idx_row != 0
                pe_row = phase + r
                for j in range(_GROUPS):
                    col = pl.ds(j * _LANES, _LANES)
                    emb = jnp.where(keep, rows_v[r, col], 0.0)
                    rows_v[r, col] = emb + pe_v[pe_row, col]
            return ()

        lax.fori_loop(0, _GROUPS, group_body, (), unroll=False)
        pltpu.sync_copy(rows_v, out_hbm.at[wid, b])
        return ()

    lax.fori_loop(0, _NCHUNK, chunk_body, (), unroll=False)


@jax.jit
def _embedding_lookup(x32, table, pe2):
    mesh = plsc.VectorSubcoreMesh(core_axis_name="c", subcore_axis_name="s")
    run = pl.kernel(
        _emb_body,
        out_type=jax.ShapeDtypeStruct((_NW, _NCHUNK, _CHUNK, _D), jnp.float32),
        mesh=mesh,
        scratch_types=[
            pltpu.VMEM((_PER_W,), jnp.int32),
            pltpu.VMEM((2 * _S, _D), jnp.float32),
            pltpu.VMEM((_CHUNK, _D), jnp.float32),
            pltpu.SemaphoreType.DMA,
        ],
        compiler_params=pltpu.CompilerParams(needs_layout_passes=False),
    )
    return run(x32, table, pe2)


def kernel(x, table):
    x32 = x.astype(jnp.int32).reshape(_NROWS)
    out = _embedding_lookup(x32, table, _pos_encoding_2x())
    return out.reshape(_B, _S, _D)


# speedup vs baseline: 2.5604x; 1.2375x over previous
import functools

import numpy as np
import jax
import jax.numpy as jnp
from jax import lax
from jax.experimental import pallas as pl
from jax.experimental.pallas import tpu as pltpu
from jax.experimental.pallas import tpu_sc as plsc

_VOCAB = 100000
_D = 128
_B = 1024
_S = 200

_NC = 2
_NS = 16
_NW = _NC * _NS
_NROWS = _B * _S
_PER_W = _NROWS // _NW
_CHUNK = 128
_NCHUNK = _PER_W // _CHUNK
_LANES = 16
_GROUPS = _CHUNK // _LANES


def _pos_encoding_2x() -> jax.Array:
    positions = np.arange(_S)
    dimensions = np.arange(_D)
    denominator = np.power(10000.0, 2 * dimensions / _D)
    angles = positions.reshape(-1, 1) / denominator.reshape(1, -1)
    pe = np.zeros(angles.shape)
    pe[:, 0::2] = np.sin(angles[:, 0::2])
    pe[:, 1::2] = np.cos(angles[:, 1::2])
    pe2 = np.concatenate([pe, pe], axis=0)
    return jnp.asarray(pe2, dtype=jnp.float32)


def _emb_body(x_hbm, table_hbm, pe_hbm, out_hbm,
              idx_v, pe_v, rows_a, rows_b, gsem_a, gsem_b, osem_a, osem_b):
    wid = lax.axis_index("s") * _NC + lax.axis_index("c")

    pltpu.sync_copy(x_hbm.at[pl.ds(wid * _PER_W, _PER_W)], idx_v)
    pltpu.sync_copy(pe_hbm, pe_v)

    bufs = (rows_a, rows_b)
    gsems = (gsem_a, gsem_b)
    osems = (osem_a, osem_b)

    def start_gather(b, buf, sem):
        idx_c = idx_v.at[pl.ds(b * _CHUNK, _CHUNK)]
        pltpu.async_copy(table_hbm.at[idx_c], buf, sem)

    def drain(sem, buf):
        pltpu.make_async_copy(out_hbm.at[wid, 0], buf, sem).wait()

    def compute(b, buf):
        phase = (b * _CHUNK) % _S

        def group_body(g, _):
            rbase = g * _LANES
            for i in range(_LANES):
                r = rbase + i
                rvec = jnp.full((_LANES,), b * _CHUNK + r, dtype=jnp.int32)
                idx_row = plsc.load_gather(idx_v, [rvec])
                keep = idx_row != 0
                pe_row = phase + r
                for j in range(_GROUPS):
                    col = pl.ds(j * _LANES, _LANES)
                    emb = jnp.where(keep, buf[r, col], 0.0)
                    buf[r, col] = emb + pe_v[pe_row, col]
            return ()

        lax.fori_loop(0, _GROUPS, group_body, (), unroll=False)

    start_gather(0, rows_a, gsem_a)

    def pair_body(p, _):
        for par in range(2):
            b = 2 * p + par
            buf, gs, osm = bufs[par], gsems[par], osems[par]
            nbuf, ngs, nosm = bufs[1 - par], gsems[1 - par], osems[1 - par]

            @pl.when(b + 1 < _NCHUNK)
            def _():
                @pl.when(b >= 1)
                def _():
                    drain(nosm, nbuf)
                start_gather(b + 1, nbuf, ngs)

            drain(gs, buf)
            compute(b, buf)
            pltpu.async_copy(buf, out_hbm.at[wid, b], osm)
        return ()

    lax.fori_loop(0, _NCHUNK // 2, pair_body, (), unroll=False)
    drain(osem_a, rows_a)
    drain(osem_b, rows_b)


@jax.jit
def _embedding_lookup(x32, table, pe2):
    mesh = plsc.VectorSubcoreMesh(core_axis_name="c", subcore_axis_name="s")
    run = pl.kernel(
        _emb_body,
        out_type=jax.ShapeDtypeStruct((_NW, _NCHUNK, _CHUNK, _D), jnp.float32),
        mesh=mesh,
        scratch_types=[
            pltpu.VMEM((_PER_W,), jnp.int32),
            pltpu.VMEM((2 * _S, _D), jnp.float32),
            pltpu.VMEM((_CHUNK, _D), jnp.float32),
            pltpu.VMEM((_CHUNK, _D), jnp.float32),
            pltpu.SemaphoreType.DMA,
            pltpu.SemaphoreType.DMA,
            pltpu.SemaphoreType.DMA,
            pltpu.SemaphoreType.DMA,
        ],
        compiler_params=pltpu.CompilerParams(needs_layout_passes=False),
    )
    return run(x32, table, pe2)


def kernel(x, table):
    x32 = x.astype(jnp.int32).reshape(_NROWS)
    out = _embedding_lookup(x32, table, _pos_encoding_2x())
    return out.reshape(_B, _S, _D)
